# Initial kernel scaffold; baseline (speedup 1.0000x reference)
import jax
import jax.numpy as jnp
from jax.experimental import pallas as pl


def kernel(x, batch, W1_pre, b1_pre, W2_pre, b2_pre, W1_post, b1_post, W2_post, b2_post):
    raise NotImplementedError("write your pallas kernel here")


# speedup vs baseline: 1.6793x; 1.6793x over previous
import functools

import jax
import jax.numpy as jnp
from jax import lax
from jax.experimental import pallas as pl
from jax.experimental.pallas import tpu as pltpu
from jax.experimental.pallas import tpu_sc as plsc

N = 320000
D = 128
G = 64
NUM_SEGMENTS = 256

GRP = 128
NGRP = N // GRP
ROW_BLK = 512
NBLK = N // ROW_BLK


def _pre_ffn_body(x_ref, w1_ref, b1_ref, w2_ref, b2_ref, out_ref):
    h = jnp.maximum(
        lax.dot_general(x_ref[...], w1_ref[...], (((1,), (0,)), ((), ())),
                        preferred_element_type=jnp.float32) + b1_ref[...],
        0.0)
    out_ref[...] = lax.dot_general(h, w2_ref[...], (((1,), (0,)), ((), ())),
                                   preferred_element_type=jnp.float32) + b2_ref[...]


def _post_ffn_body(p_ref, w1_ref, b1_ref, w2_ref, b2_ref, out_ref):
    g = p_ref[0] + p_ref[1]
    h = jnp.maximum(
        lax.dot_general(g, w1_ref[...], (((1,), (0,)), ((), ())),
                        preferred_element_type=jnp.float32) + b1_ref[...],
        0.0)
    out_ref[...] = lax.dot_general(h, w2_ref[...], (((1,), (0,)), ((), ())),
                                   preferred_element_type=jnp.float32) + b2_ref[...]


def _sc_segment_sum_body(batch2d, ftx, out, idx_v, rows_v, zeros_v, acc_sh, sem):
    nc = lax.axis_size("c")
    ns = lax.axis_size("s")
    c = lax.axis_index("c")
    s = lax.axis_index("s")
    w = c * ns + s
    nw = nc * ns

    for i in range(16):
        for j in range(D // 16):
            zeros_v[i, pl.ds(j * 16, 16)] = jnp.zeros((16,), jnp.float32)
    pltpu.sync_copy(zeros_v, acc_sh.at[pl.ds(s * 16, 16)])
    plsc.subcore_barrier()

    ng = (NGRP - w + nw - 1) // nw

    def body(i, carry):
        g = w + i * nw
        pltpu.sync_copy(batch2d.at[g], idx_v)
        pltpu.sync_copy(ftx.at[pl.ds(g * GRP, GRP)], rows_v)
        pltpu.sync_copy(rows_v, acc_sh.at[idx_v], add=True)
        return carry

    lax.fori_loop(0, ng, body, 0)
    plsc.subcore_barrier()

    @pl.when(s == 0)
    def _():
        pltpu.sync_copy(acc_sh, out.at[c])


def kernel(x, batch, W1_pre, b1_pre, W2_pre, b2_pre, W1_post, b1_post, W2_post, b2_post):
    batch2d = batch.astype(jnp.int32).reshape(NGRP, GRP)
    b1p = b1_pre.reshape(1, D)
    b2p = b2_pre.reshape(1, D)
    b1q = b1_post.reshape(1, D)
    b2q = b2_post.reshape(1, G)

    ftx = pl.pallas_call(
        _pre_ffn_body,
        grid=(NBLK,),
        in_specs=[
            pl.BlockSpec((ROW_BLK, D), lambda i: (i, 0)),
            pl.BlockSpec((D, D), lambda i: (0, 0)),
            pl.BlockSpec((1, D), lambda i: (0, 0)),
            pl.BlockSpec((D, D), lambda i: (0, 0)),
            pl.BlockSpec((1, D), lambda i: (0, 0)),
        ],
        out_specs=pl.BlockSpec((ROW_BLK, D), lambda i: (i, 0)),
        out_shape=jax.ShapeDtypeStruct((N, D), jnp.float32),
    )(x, W1_pre, b1p, W2_pre, b2p)

    mesh = plsc.VectorSubcoreMesh(core_axis_name="c", subcore_axis_name="s")
    partials = pl.kernel(
        _sc_segment_sum_body,
        out_type=jax.ShapeDtypeStruct((2, NUM_SEGMENTS, D), jnp.float32),
        mesh=mesh,
        scratch_types=[
            pltpu.VMEM((GRP,), jnp.int32),
            pltpu.VMEM((GRP, D), jnp.float32),
            pltpu.VMEM((16, D), jnp.float32),
            pltpu.VMEM_SHARED((NUM_SEGMENTS, D), jnp.float32),
            pltpu.SemaphoreType.DMA,
        ],
    )(batch2d, ftx)

    out = pl.pallas_call(
        _post_ffn_body,
        in_specs=[
            pl.BlockSpec((2, NUM_SEGMENTS, D), lambda: (0, 0, 0)),
            pl.BlockSpec((D, D), lambda: (0, 0)),
            pl.BlockSpec((1, D), lambda: (0, 0)),
            pl.BlockSpec((D, G), lambda: (0, 0)),
            pl.BlockSpec((1, G), lambda: (0, 0)),
        ],
        out_specs=pl.BlockSpec((NUM_SEGMENTS, G), lambda: (0, 0)),
        out_shape=jax.ShapeDtypeStruct((NUM_SEGMENTS, G), jnp.float32),
    )(partials, W1_post, b1q, W2_post, b2q)
    return out


# speedup vs baseline: 1.8799x; 1.1194x over previous
import functools

import jax
import jax.numpy as jnp
from jax import lax
from jax.experimental import pallas as pl
from jax.experimental.pallas import tpu as pltpu
from jax.experimental.pallas import tpu_sc as plsc

N = 320000
D = 128
G = 64
NUM_SEGMENTS = 256

GRP = 128
NGRP = N // GRP
ROW_BLK = 512
NBLK = N // ROW_BLK


def _pre_ffn_body(x_ref, w1_ref, b1_ref, w2_ref, b2_ref, out_ref):
    xb = x_ref[...].astype(jnp.bfloat16)
    h = jnp.maximum(
        lax.dot_general(xb, w1_ref[...], (((1,), (0,)), ((), ())),
                        preferred_element_type=jnp.float32) + b1_ref[...],
        0.0)
    out_ref[...] = lax.dot_general(h.astype(jnp.bfloat16), w2_ref[...],
                                   (((1,), (0,)), ((), ())),
                                   preferred_element_type=jnp.float32) + b2_ref[...]


def _post_ffn_body(p_ref, w1_ref, b1_ref, w2_ref, b2_ref, out_ref):
    g = p_ref[0] + p_ref[1]
    h = jnp.maximum(
        lax.dot_general(g, w1_ref[...], (((1,), (0,)), ((), ())),
                        preferred_element_type=jnp.float32) + b1_ref[...],
        0.0)
    out_ref[...] = lax.dot_general(h, w2_ref[...], (((1,), (0,)), ((), ())),
                                   preferred_element_type=jnp.float32) + b2_ref[...]


def _sc_segment_sum_body(batch2d, ftx, out, idx_v, rows_v, zeros_v, acc_sh,
                         sem0, sem1):
    nc = lax.axis_size("c")
    ns = lax.axis_size("s")
    c = lax.axis_index("c")
    s = lax.axis_index("s")
    w = c * ns + s
    nw = nc * ns
    sems = (sem0, sem1)

    for i in range(16):
        for j in range(D // 16):
            zeros_v[i, pl.ds(j * 16, 16)] = jnp.zeros((16,), jnp.float32)
    pltpu.sync_copy(zeros_v, acc_sh.at[pl.ds(s * 16, 16)])
    plsc.subcore_barrier()

    ng = (NGRP - w + nw - 1) // nw

    def start_group(i, b):
        g = w + i * nw
        pltpu.async_copy(batch2d.at[g], idx_v.at[b], sems[b])
        pltpu.async_copy(ftx.at[pl.ds(g * GRP, GRP)], rows_v.at[b], sems[b])

    def wait_group(b):
        pltpu.make_async_copy(batch2d.at[0], idx_v.at[b], sems[b]).wait()
        pltpu.make_async_copy(ftx.at[pl.ds(0, GRP)], rows_v.at[b], sems[b]).wait()

    @pl.when(ng > 0)
    def _():
        start_group(0, 0)

    @pl.when(ng > 1)
    def _():
        start_group(1, 1)

    def pair_body(p, carry):
        for b in range(2):
            i = 2 * p + b

            @pl.when(i < ng)
            def _():
                wait_group(b)
                pltpu.sync_copy(rows_v.at[b], acc_sh.at[idx_v.at[b]], add=True)

                @pl.when(i + 2 < ng)
                def _():
                    start_group(i + 2, b)
        return carry

    lax.fori_loop(0, (ng + 1) // 2, pair_body, 0)
    plsc.subcore_barrier()

    @pl.when(s == 0)
    def _():
        pltpu.sync_copy(acc_sh, out.at[c])


def kernel(x, batch, W1_pre, b1_pre, W2_pre, b2_pre, W1_post, b1_post, W2_post, b2_post):
    batch2d = batch.astype(jnp.int32).reshape(NGRP, GRP)
    b1p = b1_pre.reshape(1, D)
    b2p = b2_pre.reshape(1, D)
    w1_bf = W1_pre.astype(jnp.bfloat16)
    w2_bf = W2_pre.astype(jnp.bfloat16)
    b1q = b1_post.reshape(1, D)
    b2q = b2_post.reshape(1, G)

    ftx = pl.pallas_call(
        _pre_ffn_body,
        grid=(NBLK,),
        in_specs=[
            pl.BlockSpec((ROW_BLK, D), lambda i: (i, 0)),
            pl.BlockSpec((D, D), lambda i: (0, 0)),
            pl.BlockSpec((1, D), lambda i: (0, 0)),
            pl.BlockSpec((D, D), lambda i: (0, 0)),
            pl.BlockSpec((1, D), lambda i: (0, 0)),
        ],
        out_specs=pl.BlockSpec((ROW_BLK, D), lambda i: (i, 0)),
        out_shape=jax.ShapeDtypeStruct((N, D), jnp.float32),
    )(x, w1_bf, b1p, w2_bf, b2p)

    mesh = plsc.VectorSubcoreMesh(core_axis_name="c", subcore_axis_name="s")
    partials = pl.kernel(
        _sc_segment_sum_body,
        out_type=jax.ShapeDtypeStruct((2, NUM_SEGMENTS, D), jnp.float32),
        mesh=mesh,
        scratch_types=[
            pltpu.VMEM((2, GRP), jnp.int32),
            pltpu.VMEM((2, GRP, D), jnp.float32),
            pltpu.VMEM((16, D), jnp.float32),
            pltpu.VMEM_SHARED((NUM_SEGMENTS, D), jnp.float32),
            pltpu.SemaphoreType.DMA,
            pltpu.SemaphoreType.DMA,
        ],
    )(batch2d, ftx)

    out = pl.pallas_call(
        _post_ffn_body,
        in_specs=[
            pl.BlockSpec((2, NUM_SEGMENTS, D), lambda: (0, 0, 0)),
            pl.BlockSpec((D, D), lambda: (0, 0)),
            pl.BlockSpec((1, D), lambda: (0, 0)),
            pl.BlockSpec((D, G), lambda: (0, 0)),
            pl.BlockSpec((1, G), lambda: (0, 0)),
        ],
        out_specs=pl.BlockSpec((NUM_SEGMENTS, G), lambda: (0, 0)),
        out_shape=jax.ShapeDtypeStruct((NUM_SEGMENTS, G), jnp.float32),
    )(partials, W1_post, b1q, W2_post, b2q)
    return out


# speedup vs baseline: 4.0911x; 2.1763x over previous
import functools

import jax
import jax.numpy as jnp
from jax import lax
from jax.experimental import pallas as pl
from jax.experimental.pallas import tpu as pltpu
from jax.experimental.pallas import tpu_sc as plsc

N = 320000
D = 128
G = 64
NUM_SEGMENTS = 256

GRP = 128
NGRP = N // GRP
CH = 4
CGRP = NGRP // CH
CROWS = N // CH
ROW_BLK = 2000
CBLK = CROWS // ROW_BLK
NW = 32
MAXG = CGRP // NW + 1


def _pre_ffn_body(x_ref, w1_ref, b1_ref, w2_ref, b2_ref, out_ref):
    xb = x_ref[...].astype(jnp.bfloat16)
    h = jnp.maximum(
        lax.dot_general(xb, w1_ref[...], (((1,), (0,)), ((), ())),
                        preferred_element_type=jnp.float32) + b1_ref[...],
        0.0)
    out_ref[...] = lax.dot_general(h.astype(jnp.bfloat16), w2_ref[...],
                                   (((1,), (0,)), ((), ())),
                                   preferred_element_type=jnp.float32) + b2_ref[...]


def _post_ffn_body(p_ref, w1_ref, b1_ref, w2_ref, b2_ref, out_ref):
    g = jnp.sum(p_ref[...], axis=0)
    h = jnp.maximum(
        lax.dot_general(g, w1_ref[...], (((1,), (0,)), ((), ())),
                        preferred_element_type=jnp.float32) + b1_ref[...],
        0.0)
    out_ref[...] = lax.dot_general(h, w2_ref[...], (((1,), (0,)), ((), ())),
                                   preferred_element_type=jnp.float32) + b2_ref[...]


def _sc_chunk_body(idx3, ftx, out, idx_v, rows_v, zeros_v, acc_sh, semi, sem0, sem1):
    nc = lax.axis_size("c")
    ns = lax.axis_size("s")
    c = lax.axis_index("c")
    s = lax.axis_index("s")
    w = c * ns + s
    sems = (sem0, sem1)

    g0 = (CGRP * w) // NW
    g1 = (CGRP * (w + 1)) // NW
    ng = g1 - g0

    pltpu.async_copy(idx3.at[w], idx_v, semi)

    for i in range(16):
        for j in range(D // 16):
            zeros_v[i, pl.ds(j * 16, 16)] = jnp.zeros((16,), jnp.float32)
    pltpu.sync_copy(zeros_v, acc_sh.at[pl.ds(s * 16, 16)])
    pltpu.make_async_copy(idx3.at[0], idx_v, semi).wait()
    plsc.subcore_barrier()

    def start_rows(j, b):
        pltpu.async_copy(ftx.at[pl.ds((g0 + j) * GRP, GRP)], rows_v.at[b], sems[b])

    def wait_rows(b):
        pltpu.make_async_copy(ftx.at[pl.ds(0, GRP)], rows_v.at[b], sems[b]).wait()

    @pl.when(ng > 0)
    def _():
        start_rows(0, 0)

    @pl.when(ng > 1)
    def _():
        start_rows(1, 1)

    def pair_body(p, carry):
        for b in range(2):
            j = 2 * p + b

            @pl.when(j < ng)
            def _():
                wait_rows(b)
                pltpu.sync_copy(rows_v.at[b], acc_sh.at[idx_v.at[j]], add=True)

                @pl.when(j + 2 < ng)
                def _():
                    start_rows(j + 2, b)
        return carry

    lax.fori_loop(0, (ng + 1) // 2, pair_body, 0)
    plsc.subcore_barrier()

    @pl.when(s == 0)
    def _():
        pltpu.sync_copy(acc_sh, out.at[c])


def kernel(x, batch, W1_pre, b1_pre, W2_pre, b2_pre, W1_post, b1_post, W2_post, b2_post):
    batch2d = batch.astype(jnp.int32).reshape(NGRP, GRP)
    b1p = b1_pre.reshape(1, D)
    b2p = b2_pre.reshape(1, D)
    b1q = b1_post.reshape(1, D)
    b2q = b2_post.reshape(1, G)
    w1_bf = W1_pre.astype(jnp.bfloat16)
    w2_bf = W2_pre.astype(jnp.bfloat16)

    g0s = (CGRP * jnp.arange(NW, dtype=jnp.int32)) // NW
    rows = jnp.minimum(g0s[:, None] + jnp.arange(MAXG, dtype=jnp.int32)[None, :],
                       CGRP - 1)
    rows = rows[None, :, :] + CGRP * jnp.arange(CH, dtype=jnp.int32)[:, None, None]
    idx3 = batch2d[rows.reshape(-1)].reshape(CH, NW, MAXG, GRP)

    mesh = plsc.VectorSubcoreMesh(core_axis_name="c", subcore_axis_name="s",
                                  num_cores=2, num_subcores=16)
    sc_call = pl.kernel(
        _sc_chunk_body,
        out_type=jax.ShapeDtypeStruct((2, NUM_SEGMENTS, D), jnp.float32),
        mesh=mesh,
        scratch_types=[
            pltpu.VMEM((MAXG, GRP), jnp.int32),
            pltpu.VMEM((2, GRP, D), jnp.float32),
            pltpu.VMEM((16, D), jnp.float32),
            pltpu.VMEM_SHARED((NUM_SEGMENTS, D), jnp.float32),
            pltpu.SemaphoreType.DMA,
            pltpu.SemaphoreType.DMA,
            pltpu.SemaphoreType.DMA,
        ],
    )

    partials = []
    for k in range(CH):
        ftx_k = pl.pallas_call(
            _pre_ffn_body,
            grid=(CBLK,),
            in_specs=[
                pl.BlockSpec((ROW_BLK, D), functools.partial(lambda k_, i: (k_ * CBLK + i, 0), k)),
                pl.BlockSpec((D, D), lambda i: (0, 0)),
                pl.BlockSpec((1, D), lambda i: (0, 0)),
                pl.BlockSpec((D, D), lambda i: (0, 0)),
                pl.BlockSpec((1, D), lambda i: (0, 0)),
            ],
            out_specs=pl.BlockSpec((ROW_BLK, D), lambda i: (i, 0)),
            out_shape=jax.ShapeDtypeStruct((CROWS, D), jnp.float32),
        )(x, w1_bf, b1p, w2_bf, b2p)
        partials.append(sc_call(idx3[k], ftx_k))

    pstack = jnp.concatenate(partials, axis=0)

    out = pl.pallas_call(
        _post_ffn_body,
        in_specs=[
            pl.BlockSpec((2 * CH, NUM_SEGMENTS, D), lambda: (0, 0, 0)),
            pl.BlockSpec((D, D), lambda: (0, 0)),
            pl.BlockSpec((1, D), lambda: (0, 0)),
            pl.BlockSpec((D, G), lambda: (0, 0)),
            pl.BlockSpec((1, G), lambda: (0, 0)),
        ],
        out_specs=pl.BlockSpec((NUM_SEGMENTS, G), lambda: (0, 0)),
        out_shape=jax.ShapeDtypeStruct((NUM_SEGMENTS, G), jnp.float32),
    )(pstack, W1_post, b1q, W2_post, b2q)
    return out
